# speedup vs baseline: 3.4422x; 3.4422x over previous
import jax
import jax.numpy as jnp
from jax.experimental import pallas as pl
from jax.experimental.pallas import tpu as pltpu

N = 16384
B = 16
TILE = 2048
GRID = N // TILE
Z128 = 128


def _body(z_ref, pos_ref, b_ref, emb_ref, Wp_ref, W1_ref, b1_ref, W2_ref,
          b2_ref, out_ref, A_sc, P_sc, acc_sc, cnt_sc):
    i = pl.program_id(0)

    @pl.when(i == 0)
    def _init():
        A_sc[...] = jnp.dot(emb_ref[...], W1_ref[...],
                            preferred_element_type=jnp.float32)
        P_sc[...] = jnp.dot(Wp_ref[...], W1_ref[...],
                            preferred_element_type=jnp.float32)
        acc_sc[...] = jnp.zeros_like(acc_sc)
        cnt_sc[...] = jnp.zeros_like(cnt_sc)

    zc = z_ref[0]
    lane = jax.lax.broadcasted_iota(jnp.int32, (TILE, Z128), 1)
    oh_z = (zc == lane).astype(jnp.float32)
    a = jnp.dot(oh_z, A_sc[...], preferred_element_type=jnp.float32)
    p = jnp.dot(pos_ref[...], P_sc[...], preferred_element_type=jnp.float32)
    hpre = a + p + b1_ref[...]
    h = hpre * jax.nn.sigmoid(hpre)

    br = b_ref[0]
    seg = jax.lax.broadcasted_iota(jnp.int32, (B, TILE), 0)
    ohbT = (seg == br).astype(jnp.float32)
    acc_sc[...] += jnp.dot(ohbT, h, preferred_element_type=jnp.float32)
    cnt_sc[...] += jnp.sum(ohbT, axis=1, keepdims=True)

    @pl.when(i == GRID - 1)
    def _fin():
        out_ref[...] = (jnp.dot(acc_sc[...], W2_ref[...],
                                preferred_element_type=jnp.float32)
                        + cnt_sc[...] * b2_ref[...])


def kernel(z, pos, batch, emb, Wp, W1, b1, W2, b2):
    D = emb.shape[1]
    H = W1.shape[1]
    emb_p = jnp.pad(emb, ((0, Z128 - emb.shape[0]), (0, 0)))
    pos_p = jnp.pad(pos, ((0, 0), (0, 5)))
    Wp_p = jnp.pad(Wp, ((0, 5), (0, 0)))
    z_in = z.reshape(GRID, TILE, 1).astype(jnp.int32)
    b_in = batch.reshape(GRID, 1, TILE).astype(jnp.int32)
    b1r = b1.reshape(1, H)
    b2r = b2.reshape(1, 1)

    out = pl.pallas_call(
        _body,
        grid=(GRID,),
        in_specs=[
            pl.BlockSpec((1, TILE, 1), lambda i: (i, 0, 0)),
            pl.BlockSpec((TILE, 8), lambda i: (i, 0)),
            pl.BlockSpec((1, 1, TILE), lambda i: (i, 0, 0)),
            pl.BlockSpec((Z128, D), lambda i: (0, 0)),
            pl.BlockSpec((8, D), lambda i: (0, 0)),
            pl.BlockSpec((D, H), lambda i: (0, 0)),
            pl.BlockSpec((1, H), lambda i: (0, 0)),
            pl.BlockSpec((H, 1), lambda i: (0, 0)),
            pl.BlockSpec((1, 1), lambda i: (0, 0)),
        ],
        out_specs=pl.BlockSpec((B, 1), lambda i: (0, 0)),
        out_shape=jax.ShapeDtypeStruct((B, 1), jnp.float32),
        scratch_shapes=[
            pltpu.VMEM((Z128, H), jnp.float32),
            pltpu.VMEM((8, H), jnp.float32),
            pltpu.VMEM((B, H), jnp.float32),
            pltpu.VMEM((B, 1), jnp.float32),
        ],
    )(z_in, pos_p, b_in, emb_p, Wp_p, W1, b1r, W2, b2r)
    return out
